# baseline (speedup 1.0000x reference)
import functools
import math

import jax
import jax.numpy as jnp
from jax.experimental import pallas as pl
from jax.experimental.pallas import tpu as pltpu
from jax.experimental.pallas import tpu_sc as plsc

M = 0.4
H = 0.333
S = 10.0
EPS = 0.001
LANES = 128
UNROLL = 4

SPLIT = 512
SC_CHUNK = 2048
SC_GROUPS = SC_CHUNK // 16


def _stream_kernel(label_ref, cos_ref, acc_ref, lab_ref, *, n_cols, block_r):
    iota = jax.lax.broadcasted_iota(jnp.int32, (block_r, LANES), 1)
    lab = label_ref[...]
    labb = jnp.broadcast_to(lab, (block_r, LANES))

    n_full = n_cols // LANES
    n_loop = (n_full // UNROLL) * UNROLL

    def chunk(col0, e_acc, l_acc):
        c = jnp.clip(cos_ref[:, pl.ds(col0, LANES)], -1.0 + EPS, 1.0 - EPS)
        e_acc = e_acc + jnp.exp(c * S - S)
        l_acc = l_acc + jnp.where(iota + col0 == labb, c, 0.0)
        return e_acc, l_acc

    def body(i, carry):
        e_acc, l_acc = carry
        base = i * (LANES * UNROLL)
        for u in range(UNROLL):
            e_acc, l_acc = chunk(base + u * LANES, e_acc, l_acc)
        return e_acc, l_acc

    zeros = jnp.zeros((block_r, LANES), jnp.float32)
    e_acc, l_acc = jax.lax.fori_loop(0, n_loop // UNROLL, body, (zeros, zeros))

    for col0 in range(n_loop * LANES, n_full * LANES, LANES):
        e_acc, l_acc = chunk(col0, e_acc, l_acc)

    rem = n_cols - n_full * LANES
    if rem:
        col0 = n_cols - LANES
        keep = iota >= (LANES - rem)
        c = jnp.clip(cos_ref[:, pl.ds(col0, LANES)], -1.0 + EPS, 1.0 - EPS)
        e_acc = e_acc + jnp.where(keep, jnp.exp(c * S - S), 0.0)
        l_acc = l_acc + jnp.where(keep & (iota + col0 == labb), c, 0.0)

    acc_ref[...] = e_acc
    lab_ref[...] = l_acc


def _make_sc_stream(n_rows, n_cols):
    mesh = plsc.VectorSubcoreMesh(core_axis_name="c", subcore_axis_name="s")
    sc_rows = n_rows - SPLIT
    trs_per_w = sc_rows // (32 * 8)
    rows_per_w = trs_per_w * 8
    n_chunks = n_cols // SC_CHUNK
    n_pairs = n_chunks // 2
    last_full = (n_chunks - 1) * SC_CHUNK
    t1_col = n_chunks * SC_CHUNK
    t1_w = (n_cols // LANES) * LANES - t1_col
    t2_col = (n_cols // LANES) * LANES
    t2_w = n_cols - t2_col

    @functools.partial(
        pl.kernel, mesh=mesh,
        out_type=[
            jax.ShapeDtypeStruct((sc_rows * 16,), jnp.float32),
            jax.ShapeDtypeStruct((sc_rows,), jnp.float32),
        ],
        scratch_types=[
            pltpu.VMEM((8, SC_CHUNK), jnp.float32),
            pltpu.VMEM((8, SC_CHUNK), jnp.float32),
            pltpu.VMEM((8, t1_w), jnp.float32),
            pltpu.VMEM((8, t2_w), jnp.float32),
            pltpu.VMEM((16 * 16,), jnp.float32),
            pltpu.VMEM((16,), jnp.float32),
            pltpu.VMEM((16,), jnp.float32),
            pltpu.SemaphoreType.DMA,
            pltpu.SemaphoreType.DMA,
        ],
    )
    def sc_stream(cos_hbm, label_hbm, sum_hbm, lab_hbm,
                  buf0, buf1, tb1, tb2, sum_v, lab_v, lab_s, sem0, sem1):
        cid = jax.lax.axis_index("c")
        sid = jax.lax.axis_index("s")
        w = sid * 2 + cid
        iota16 = jax.lax.iota(jnp.int32, 16)

        def row_block(buf, rr, acc, labacc, rel, n_groups, unroll):
            def gbody(v, carry):
                a, la, r = carry
                g0 = v * (16 * unroll)
                for u in range(unroll):
                    c = jnp.clip(buf[rr, pl.ds(g0 + u * 16, 16)],
                                 -1.0 + EPS, 1.0 - EPS)
                    a = a + jnp.exp(c * S - S)
                    la = jnp.where(iota16 == r, c, la)
                    r = r - 16
                return a, la, r
            return jax.lax.fori_loop(0, n_groups // unroll, gbody,
                                     (acc, labacc, rel))

        def start(col, buf, sem):
            pltpu.async_copy(
                cos_hbm.at[pl.ds(row0, 8),
                           pl.ds(pl.multiple_of(col, LANES), SC_CHUNK)],
                buf, sem)

        def wait(buf, sem):
            pltpu.make_async_copy(
                cos_hbm.at[pl.ds(row0, 8), pl.ds(0, SC_CHUNK)], buf, sem).wait()

        pltpu.sync_copy(label_hbm.at[pl.ds(SPLIT + w * rows_per_w, rows_per_w)],
                        lab_s)
        labvec = jnp.zeros((16,), jnp.float32)

        for t in range(trs_per_w):
            row0 = SPLIT + w * rows_per_w + t * 8

            start(0, buf0, sem0)
            start(SC_CHUNK, buf1, sem1)

            accs = [jnp.zeros((16,), jnp.float32) for _ in range(8)]
            labaccs = [jnp.zeros((16,), jnp.float32) for _ in range(8)]
            labv = lab_s[...]
            rels = [labv.at[jnp.full((16,), t * 8 + rr, jnp.int32)]
                    .get(mode="promise_in_bounds").astype(jnp.int32)
                    for rr in range(8)]

            def pbody(j, carry):
                accs, labaccs, rels = (list(x) for x in carry)
                wait(buf0, sem0)
                for rr in range(8):
                    accs[rr], labaccs[rr], rels[rr] = row_block(
                        buf0, rr, accs[rr], labaccs[rr], rels[rr],
                        SC_GROUPS, UNROLL)
                start(jnp.minimum((2 * j + 2) * SC_CHUNK, last_full), buf0, sem0)
                wait(buf1, sem1)
                for rr in range(8):
                    accs[rr], labaccs[rr], rels[rr] = row_block(
                        buf1, rr, accs[rr], labaccs[rr], rels[rr],
                        SC_GROUPS, UNROLL)
                start(jnp.minimum((2 * j + 3) * SC_CHUNK, last_full), buf1, sem1)
                return tuple(accs), tuple(labaccs), tuple(rels)

            accs, labaccs, rels = jax.lax.fori_loop(
                0, n_pairs, pbody,
                (tuple(accs), tuple(labaccs), tuple(rels)))
            accs, labaccs, rels = list(accs), list(labaccs), list(rels)

            wait(buf0, sem0)
            wait(buf1, sem1)
            pltpu.async_copy(
                cos_hbm.at[pl.ds(row0, 8), pl.ds(t1_col, t1_w)], tb1, sem0)
            pltpu.async_copy(
                cos_hbm.at[pl.ds(row0, 8), pl.ds(t2_col, t2_w)], tb2, sem1)
            pltpu.make_async_copy(
                cos_hbm.at[pl.ds(row0, 8), pl.ds(t1_col, t1_w)], tb1,
                sem0).wait()
            for rr in range(8):
                accs[rr], labaccs[rr], rels[rr] = row_block(
                    tb1, rr, accs[rr], labaccs[rr], rels[rr], t1_w // 16,
                    UNROLL)
            pltpu.make_async_copy(
                cos_hbm.at[pl.ds(row0, 8), pl.ds(t2_col, t2_w)], tb2,
                sem1).wait()
            for rr in range(8):
                accs[rr], labaccs[rr], rels[rr] = row_block(
                    tb2, rr, accs[rr], labaccs[rr], rels[rr], t2_w // 16, 1)

            for rr in range(8):
                sum_v[pl.ds((t * 8 + rr) * 16, 16)] = accs[rr]
                allv = labaccs[rr]
                for sh in (1, 2, 4, 8):
                    allv = allv + allv.at[(iota16 + sh) % 16].get(
                        mode="promise_in_bounds")
                labvec = jnp.where(iota16 == t * 8 + rr, allv, labvec)

        lab_v[...] = labvec
        pltpu.sync_copy(
            sum_v, sum_hbm.at[pl.ds(w * rows_per_w * 16, rows_per_w * 16)])
        pltpu.sync_copy(lab_v, lab_hbm.at[pl.ds(w * rows_per_w, rows_per_w)])

    return sc_stream


def _combine_kernel(norms_ref, acc_ref, lab_ref, scsum_ref, sclab_ref,
                    loss_ref, *, n_rows):
    n = jnp.clip(norms_ref[...], 0.001, 100.0)
    mean = jnp.mean(n)
    var = jnp.sum((n - mean) ** 2) / (n_rows - 1)
    std = jnp.sqrt(var)
    ms = jnp.clip((n - mean) / (std + EPS) * H, -1.0, 1.0)
    g = -M * ms
    g_add = M + M * ms

    sumexp = jnp.concatenate(
        [jnp.sum(acc_ref[...], axis=1, keepdims=True),
         jnp.sum(scsum_ref[...], axis=1, keepdims=True)], axis=0)
    c_lab = jnp.concatenate(
        [jnp.sum(lab_ref[...], axis=1, keepdims=True),
         jnp.clip(sclab_ref[...], -1.0 + EPS, 1.0 - EPS)], axis=0)

    z_mid = c_lab * jnp.cos(g) - jnp.sqrt(1.0 - c_lab * c_lab) * jnp.sin(g)
    lo = (g < EPS) & (c_lab > jnp.cos(EPS - g))
    hi = (g > -EPS) & (c_lab < -jnp.cos(EPS + g))
    z_clipped = jnp.where(lo, math.cos(EPS),
                          jnp.where(hi, math.cos(math.pi - EPS), z_mid))
    z_new = z_clipped - g_add
    total = sumexp - jnp.exp(S * c_lab - S) + jnp.exp(S * z_new - S)
    loss_i = jnp.log(total) + S - S * z_new
    loss_ref[...] = jnp.mean(loss_i, axis=(0, 1), keepdims=True)


@jax.jit
def kernel(cosine, norms, label):
    n_rows, n_cols = cosine.shape
    label1d = label.astype(jnp.int32)
    label2d = label1d.reshape(n_rows, 1)
    block_r = 64
    num_i = SPLIT // block_r

    acc, lab = pl.pallas_call(
        functools.partial(_stream_kernel, n_cols=n_cols, block_r=block_r),
        grid=(num_i,),
        in_specs=[
            pl.BlockSpec((block_r, 1), lambda i: (i, 0)),
            pl.BlockSpec((block_r, n_cols), lambda i: (i, 0)),
        ],
        out_specs=[
            pl.BlockSpec((block_r, LANES), lambda i: (i, 0)),
            pl.BlockSpec((block_r, LANES), lambda i: (i, 0)),
        ],
        out_shape=[
            jax.ShapeDtypeStruct((SPLIT, LANES), jnp.float32),
            jax.ShapeDtypeStruct((SPLIT, LANES), jnp.float32),
        ],
    )(label2d, cosine)

    sc_sum, sc_lab = _make_sc_stream(n_rows, n_cols)(
        cosine, label1d.astype(jnp.float32))

    loss = pl.pallas_call(
        functools.partial(_combine_kernel, n_rows=n_rows),
        out_shape=jax.ShapeDtypeStruct((1, 1), jnp.float32),
    )(norms, acc, lab,
      sc_sum.reshape(n_rows - SPLIT, 16), sc_lab.reshape(n_rows - SPLIT, 1))
    return loss[0, 0]
